# baseline (speedup 1.0000x reference)
import jax
import jax.numpy as jnp
from jax.experimental import pallas as pl

_BLOCK_ROWS = 2048


def _copy_body(in0_ref, in1_ref, out0_ref, out1_ref):
    out0_ref[...] = in0_ref[...]
    out1_ref[...] = in1_ref[...]


def kernel(input0, input1, input2, input3):
    M, B = input0.shape
    spec = pl.BlockSpec((_BLOCK_ROWS, B), lambda i: (i, 0))
    out0, out1 = pl.pallas_call(
        _copy_body,
        grid=(M // _BLOCK_ROWS,),
        in_specs=[spec, spec],
        out_specs=[spec, spec],
        out_shape=[
            jax.ShapeDtypeStruct((M, B), input0.dtype),
            jax.ShapeDtypeStruct((M, B), input1.dtype),
        ],
    )(input0, input1)
    return (out0, out1)


# speedup vs baseline: 1.0173x; 1.0173x over previous
import jax
import jax.numpy as jnp
from jax.experimental import pallas as pl

_BLOCK_ROWS = 4096


def _copy_body(in0_ref, in1_ref, out0_ref, out1_ref):
    out0_ref[...] = in0_ref[...]
    out1_ref[...] = in1_ref[...]


def kernel(input0, input1, input2, input3):
    M, B = input0.shape
    spec = pl.BlockSpec((_BLOCK_ROWS, B), lambda i: (i, 0))
    out0, out1 = pl.pallas_call(
        _copy_body,
        grid=(M // _BLOCK_ROWS,),
        in_specs=[spec, spec],
        out_specs=[spec, spec],
        out_shape=[
            jax.ShapeDtypeStruct((M, B), input0.dtype),
            jax.ShapeDtypeStruct((M, B), input1.dtype),
        ],
    )(input0, input1)
    return (out0, out1)
